# baseline (speedup 1.0000x reference)
import dataclasses
import functools

import jax
import jax.numpy as jnp
from jax import lax
from jax.experimental import pallas as pl
from jax.experimental.pallas import tpu as pltpu
from jax.experimental.pallas import tpu_sc as plsc

N_NODES = 10000
N_EDGES = 320000
D_IN = 128
H = 64

NC = 2
NS = 16
L = 16
NW = NC * NS
EPW = N_EDGES // NW
BLK = 80
N_PAD = 10112
STRIPE = N_PAD // NS

_vmesh = plsc.VectorSubcoreMesh(
    core_axis_name="c", subcore_axis_name="s", num_cores=NC, num_subcores=NS)

_sc_params = pltpu.CompilerParams(
    needs_layout_passes=False, use_tc_tiling_on_sc=False)


NH = N_NODES // 2


def _tc1_body(xq_ref, xa_ref, wq_ref, bq_ref, wa_ref, ba_ref,
              wl_ref, bl_ref, lg_ref,
              t_ref, zq_ref, wa_out_ref, ew_ref):
    wq = wq_ref[...]
    wa = wa_ref[...]
    wl = wl_ref[...]
    z64 = jnp.zeros((H, H), jnp.float32)
    wl2 = jnp.concatenate(
        [jnp.concatenate([wl, z64], axis=1),
         jnp.concatenate([z64, wl], axis=1)], axis=0)
    bb = jnp.concatenate([bl_ref[...], bl_ref[...]], axis=1)
    h_qt = jnp.maximum(jnp.dot(xq_ref[0:NH, :], wq,
                               preferred_element_type=jnp.float32)
                       + bq_ref[...], 0.0)
    h_qb = jnp.maximum(jnp.dot(xq_ref[NH:, :], wq,
                               preferred_element_type=jnp.float32)
                       + bq_ref[...], 0.0)
    h128 = jnp.concatenate([h_qt, h_qb], axis=1)
    t128 = jnp.dot(h128, wl2, preferred_element_type=jnp.float32)
    t_ref[...] = t128
    zq_ref[...] = t128 + bb

    h_at = jnp.maximum(jnp.dot(xa_ref[0:NH, :], wa,
                               preferred_element_type=jnp.float32)
                       + ba_ref[...], 0.0)
    h_ab = jnp.maximum(jnp.dot(xa_ref[NH:, :], wa,
                               preferred_element_type=jnp.float32)
                       + ba_ref[...], 0.0)
    a128 = jnp.concatenate([h_at, h_ab], axis=1)
    wa_out_ref[...] = (jnp.dot(a128, wl2,
                               preferred_element_type=jnp.float32)
                       + bb)

    ew_ref[...] = jax.nn.sigmoid(lg_ref[...])


def _tc1(xq, xa, wq, bq, wa, ba, wl, bl, lg2d):
    return pl.pallas_call(
        _tc1_body,
        out_shape=(
            jax.ShapeDtypeStruct((NH, 2 * H), jnp.float32),
            jax.ShapeDtypeStruct((NH, 2 * H), jnp.float32),
            jax.ShapeDtypeStruct((NH, 2 * H), jnp.float32),
            jax.ShapeDtypeStruct(lg2d.shape, jnp.float32),
        ),
    )(xq, xa, wq, bq, wa, ba, wl, bl, lg2d)


NBUF = 5
NBLK = EPW // BLK


def _sc_msg_body(hq_hbm, src_hbm, dst_hbm, ew_hbm, zeros_hbm, agg_hbm,
                 src_v, dst_v, ew_v,
                 r0, r1, r2, r3, r4,
                 agg_sh,
                 g0s, g1s, g2s, g3s, g4s,
                 s0s, s1s, s2s, s3s, s4s):
    cid = lax.axis_index("c")
    sid = lax.axis_index("s")
    wid = sid * NC + cid
    rows = (r0, r1, r2, r3, r4)
    gsem = (g0s, g1s, g2s, g3s, g4s)
    ssem = (s0s, s1s, s2s, s3s, s4s)

    pltpu.sync_copy(zeros_hbm.at[pl.ds(sid * STRIPE, STRIPE)],
                    agg_sh.at[pl.ds(sid * STRIPE, STRIPE)])
    pltpu.sync_copy(src_hbm.at[wid], src_v)
    pltpu.sync_copy(dst_hbm.at[wid], dst_v)
    pltpu.sync_copy(ew_hbm.at[wid], ew_v)
    plsc.subcore_barrier()

    def issue_gather(g, b):
        pltpu.async_copy(hq_hbm.at[src_v.at[pl.ds(g * BLK, BLK)]],
                         rows[b], gsem[b])

    for b in range(3):
        issue_gather(b, b)

    @pl.loop(0, NBLK, step=NBUF)
    def _(blk0):
        for b in range(NBUF):
            g = blk0 + b
            pltpu.make_async_copy(hq_hbm.at[src_v.at[pl.ds(g * BLK, BLK)]],
                                  rows[b], gsem[b]).wait()

            @plsc.parallel_loop(0, BLK, unroll=8)
            def _(i):
                wv = plsc.load_gather(
                    ew_v, [jnp.full((L,), g * BLK + i, jnp.int32)])
                for j in range(H // L):
                    sl = pl.ds(j * L, L)
                    rows[b][i, sl] = rows[b][i, sl] * wv

            pltpu.async_copy(rows[b], agg_sh.at[dst_v.at[g]],
                             ssem[b], add=True)

            h = g + 3
            bh = (b + 3) % NBUF

            @pl.when(jnp.logical_and(h >= NBUF, h < NBLK))
            def _():
                pltpu.make_async_copy(rows[bh], agg_sh.at[dst_v.at[h]],
                                      ssem[bh]).wait()

            @pl.when(h < NBLK)
            def _():
                issue_gather(h, bh)

    for b in range(NBUF):
        pltpu.make_async_copy(rows[b], agg_sh.at[dst_v.at[0]], ssem[b]).wait()
    plsc.subcore_barrier()
    pltpu.sync_copy(agg_sh.at[pl.ds(sid * STRIPE, STRIPE)],
                    agg_hbm.at[cid].at[pl.ds(sid * STRIPE, STRIPE)])


def _sc_msg(hq, src, dst, ew, zeros):
    kern = functools.partial(
        pl.kernel,
        out_type=jax.ShapeDtypeStruct((NC, N_PAD, H), jnp.float32),
        mesh=_vmesh,
        scratch_types=(
            [pltpu.VMEM((EPW,), jnp.int32),
             pltpu.VMEM((NBLK, BLK), jnp.int32),
             pltpu.VMEM((EPW,), jnp.float32)]
            + [pltpu.VMEM((BLK, H), jnp.float32) for _ in range(NBUF)]
            + [pltpu.VMEM_SHARED((N_PAD, H), jnp.float32)]
            + [pltpu.SemaphoreType.DMA for _ in range(2 * NBUF)]
        ),
        compiler_params=_sc_params,
    )(_sc_msg_body)
    return kern(hq, src, dst, ew, zeros)


def _tc2_body(wa_ref, aggs_ref, za_ref):
    za_ref[...] = (wa_ref[...] + aggs_ref[0, 0:NH, :]
                   + aggs_ref[1, 0:NH, :])


def _tc2(wa128, aggs128):
    return pl.pallas_call(
        _tc2_body,
        out_shape=jax.ShapeDtypeStruct((NH, 2 * H), jnp.float32),
    )(wa128, aggs128)


def _sc_dec_body(zq_hbm, za_hbm, qi_hbm, ai_hbm, pred_hbm,
                 qi_v, ai_v,
                 q0, q1, q2, q3, q4, a0, a1, a2, a3, a4,
                 tmp_v, out_v,
                 sq0, sq1, sq2, sq3, sq4, sa0, sa1, sa2, sa3, sa4):
    cid = lax.axis_index("c")
    sid = lax.axis_index("s")
    wid = sid * NC + cid
    base = wid * EPW
    qrs = (q0, q1, q2, q3, q4)
    ars = (a0, a1, a2, a3, a4)
    qsem = (sq0, sq1, sq2, sq3, sq4)
    asem = (sa0, sa1, sa2, sa3, sa4)

    pltpu.sync_copy(qi_hbm.at[wid], qi_v)
    pltpu.sync_copy(ai_hbm.at[wid], ai_v)

    def issue_gather(g, b):
        sl = pl.ds(g * BLK, BLK)
        pltpu.async_copy(zq_hbm.at[qi_v.at[sl]], qrs[b], qsem[b])
        pltpu.async_copy(za_hbm.at[ai_v.at[sl]], ars[b], asem[b])

    for b in range(3):
        issue_gather(b, b)

    @pl.loop(0, NBLK, step=NBUF)
    def _(blk0):
        for b in range(NBUF):
            g = blk0 + b
            sl = pl.ds(g * BLK, BLK)
            pltpu.make_async_copy(zq_hbm.at[qi_v.at[sl]],
                                  qrs[b], qsem[b]).wait()
            pltpu.make_async_copy(za_hbm.at[ai_v.at[sl]],
                                  ars[b], asem[b]).wait()

            @plsc.parallel_loop(0, BLK, unroll=8)
            def _(i):
                acc = qrs[b][i, pl.ds(0, L)] * ars[b][i, pl.ds(0, L)]
                for j in range(1, H // L):
                    slj = pl.ds(j * L, L)
                    acc = acc + qrs[b][i, slj] * ars[b][i, slj]
                tmp_v[pl.ds(i * L, L)] = acc

            iota = lax.iota(jnp.int32, L)
            for grp in range(BLK // L):
                res = jnp.zeros((L,), jnp.float32)
                for k in range(L):
                    idx = iota * L + (grp * L * L + k)
                    res = res + plsc.load_gather(tmp_v, [idx])
                res = 1.0 / (1.0 + jnp.exp(-res))
                out_v[pl.ds(grp * L, L)] = res

            pltpu.sync_copy(out_v, pred_hbm.at[pl.ds(base + g * BLK, BLK)])

            h = g + 3

            @pl.when(h < NBLK)
            def _():
                issue_gather(h, (b + 3) % NBUF)


def _sc_dec(zq, za, qi, ai):
    kern = functools.partial(
        pl.kernel,
        out_type=jax.ShapeDtypeStruct((N_EDGES,), jnp.float32),
        mesh=_vmesh,
        scratch_types=(
            [pltpu.VMEM((EPW,), jnp.int32),
             pltpu.VMEM((EPW,), jnp.int32)]
            + [pltpu.VMEM((BLK, H), jnp.float32) for _ in range(2 * NBUF)]
            + [pltpu.VMEM((BLK * L,), jnp.float32),
               pltpu.VMEM((BLK,), jnp.float32)]
            + [pltpu.SemaphoreType.DMA for _ in range(2 * NBUF)]
        ),
        compiler_params=_sc_params,
    )(_sc_dec_body)
    return kern(zq, za, qi, ai)


def _renumber(i):
    return jnp.where(i < NH, 2 * i, 2 * i - (N_NODES - 1))


def kernel(x_question, x_answer, edge_index, edge_label_index,
           Wq, bq, Wa, ba, W_lin, b_lin, edge_weight_logits):
    lg2d = edge_weight_logits.reshape(N_EDGES // D_IN, D_IN)

    t128, zq128, wa128, ew2d = _tc1(
        x_question, x_answer, Wq, bq.reshape(1, H), Wa, ba.reshape(1, H),
        W_lin, b_lin.reshape(1, H), lg2d)
    ew = ew2d.reshape(NW, EPW)

    src = _renumber(edge_index[0]).reshape(NW, EPW)
    dst = _renumber(edge_index[1]).reshape(NW, NBLK, BLK)
    zeros = jnp.zeros((N_PAD, H), jnp.float32)
    aggs = _sc_msg(t128.reshape(N_NODES, H), src, dst, ew, zeros)

    za128 = _tc2(wa128, aggs.reshape(NC, N_PAD // 2, 2 * H))

    return _sc_dec(zq128.reshape(N_NODES, H), za128.reshape(N_NODES, H),
                   _renumber(edge_label_index[0]).reshape(NW, EPW),
                   _renumber(edge_label_index[1]).reshape(NW, EPW))


# speedup vs baseline: 1.0531x; 1.0531x over previous
import dataclasses
import functools

import jax
import jax.numpy as jnp
from jax import lax
from jax.experimental import pallas as pl
from jax.experimental.pallas import tpu as pltpu
from jax.experimental.pallas import tpu_sc as plsc

N_NODES = 10000
N_EDGES = 320000
D_IN = 128
H = 64

NC = 2
NS = 16
L = 16
NW = NC * NS
EPW = N_EDGES // NW
BLK = 80
N_PAD = 10112
STRIPE = N_PAD // NS

_vmesh = plsc.VectorSubcoreMesh(
    core_axis_name="c", subcore_axis_name="s", num_cores=NC, num_subcores=NS)

_sc_params = pltpu.CompilerParams(
    needs_layout_passes=False, use_tc_tiling_on_sc=False)


NH = N_NODES // 2


def _tc1_body(xq_ref, xa_ref, wq_ref, bq_ref, wa_ref, ba_ref,
              wl_ref, bl_ref, lg_ref,
              t_ref, zq_ref, wa_out_ref, ew_ref):
    wq = wq_ref[...]
    wa = wa_ref[...]
    wl = wl_ref[...]
    z64 = jnp.zeros((H, H), jnp.float32)
    wl2 = jnp.concatenate(
        [jnp.concatenate([wl, z64], axis=1),
         jnp.concatenate([z64, wl], axis=1)], axis=0)
    bb = jnp.concatenate([bl_ref[...], bl_ref[...]], axis=1)
    h_qt = jnp.maximum(jnp.dot(xq_ref[0:NH, :], wq,
                               preferred_element_type=jnp.float32)
                       + bq_ref[...], 0.0)
    h_qb = jnp.maximum(jnp.dot(xq_ref[NH:, :], wq,
                               preferred_element_type=jnp.float32)
                       + bq_ref[...], 0.0)
    h128 = jnp.concatenate([h_qt, h_qb], axis=1)
    t128 = jnp.dot(h128, wl2, preferred_element_type=jnp.float32)
    t_ref[...] = t128
    zq_ref[...] = t128 + bb

    h_at = jnp.maximum(jnp.dot(xa_ref[0:NH, :], wa,
                               preferred_element_type=jnp.float32)
                       + ba_ref[...], 0.0)
    h_ab = jnp.maximum(jnp.dot(xa_ref[NH:, :], wa,
                               preferred_element_type=jnp.float32)
                       + ba_ref[...], 0.0)
    a128 = jnp.concatenate([h_at, h_ab], axis=1)
    wa_out_ref[...] = (jnp.dot(a128, wl2,
                               preferred_element_type=jnp.float32)
                       + bb)

    ew_ref[...] = jax.nn.sigmoid(lg_ref[...])


def _tc1(xq, xa, wq, bq, wa, ba, wl, bl, lg2d):
    return pl.pallas_call(
        _tc1_body,
        out_shape=(
            jax.ShapeDtypeStruct((NH, 2 * H), jnp.float32),
            jax.ShapeDtypeStruct((NH, 2 * H), jnp.float32),
            jax.ShapeDtypeStruct((NH, 2 * H), jnp.float32),
            jax.ShapeDtypeStruct(lg2d.shape, jnp.float32),
        ),
    )(xq, xa, wq, bq, wa, ba, wl, bl, lg2d)


NBUF = 5
NBLK = EPW // BLK


def _sc_msg_body(hq_hbm, src_hbm, dst_hbm, ew_hbm, zeros_hbm, agg_hbm,
                 src_v, dst_v, ew_v,
                 r0, r1, r2, r3, r4,
                 agg_sh,
                 g0s, g1s, g2s, g3s, g4s,
                 s0s, s1s, s2s, s3s, s4s):
    cid = lax.axis_index("c")
    sid = lax.axis_index("s")
    wid = sid * NC + cid
    rows = (r0, r1, r2, r3, r4)
    gsem = (g0s, g1s, g2s, g3s, g4s)
    ssem = (s0s, s1s, s2s, s3s, s4s)

    pltpu.sync_copy(zeros_hbm.at[pl.ds(sid * STRIPE, STRIPE)],
                    agg_sh.at[pl.ds(sid * STRIPE, STRIPE)])
    pltpu.sync_copy(src_hbm.at[wid], src_v)
    pltpu.sync_copy(dst_hbm.at[wid], dst_v)
    pltpu.sync_copy(ew_hbm.at[wid], ew_v)
    plsc.subcore_barrier()

    def issue_gather(g, b):
        pltpu.async_copy(hq_hbm.at[src_v.at[pl.ds(g * BLK, BLK)]],
                         rows[b], gsem[b])

    for b in range(3):
        issue_gather(b, b)

    @pl.loop(0, NBLK, step=NBUF)
    def _(blk0):
        for b in range(NBUF):
            g = blk0 + b
            pltpu.make_async_copy(hq_hbm.at[src_v.at[pl.ds(g * BLK, BLK)]],
                                  rows[b], gsem[b]).wait()

            @plsc.parallel_loop(0, BLK, unroll=4)
            def _(i):
                wv = plsc.load_gather(
                    ew_v, [jnp.full((L,), g * BLK + i, jnp.int32)])
                for j in range(H // L):
                    sl = pl.ds(j * L, L)
                    rows[b][i, sl] = rows[b][i, sl] * wv

            pltpu.async_copy(rows[b], agg_sh.at[dst_v.at[g]],
                             ssem[b], add=True)

            h = g + 3
            bh = (b + 3) % NBUF

            @pl.when(jnp.logical_and(h >= NBUF, h < NBLK))
            def _():
                pltpu.make_async_copy(rows[bh], agg_sh.at[dst_v.at[h]],
                                      ssem[bh]).wait()

            @pl.when(h < NBLK)
            def _():
                issue_gather(h, bh)

    for b in range(NBUF):
        pltpu.make_async_copy(rows[b], agg_sh.at[dst_v.at[0]], ssem[b]).wait()
    plsc.subcore_barrier()
    pltpu.sync_copy(agg_sh.at[pl.ds(sid * STRIPE, STRIPE)],
                    agg_hbm.at[cid].at[pl.ds(sid * STRIPE, STRIPE)])


def _sc_msg(hq, src, dst, ew, zeros):
    kern = functools.partial(
        pl.kernel,
        out_type=jax.ShapeDtypeStruct((NC, N_PAD, H), jnp.float32),
        mesh=_vmesh,
        scratch_types=(
            [pltpu.VMEM((EPW,), jnp.int32),
             pltpu.VMEM((NBLK, BLK), jnp.int32),
             pltpu.VMEM((EPW,), jnp.float32)]
            + [pltpu.VMEM((BLK, H), jnp.float32) for _ in range(NBUF)]
            + [pltpu.VMEM_SHARED((N_PAD, H), jnp.float32)]
            + [pltpu.SemaphoreType.DMA for _ in range(2 * NBUF)]
        ),
        compiler_params=_sc_params,
    )(_sc_msg_body)
    return kern(hq, src, dst, ew, zeros)


def _tc2_body(wa_ref, aggs_ref, za_ref):
    za_ref[...] = (wa_ref[...] + aggs_ref[0, 0:NH, :]
                   + aggs_ref[1, 0:NH, :])


def _tc2(wa128, aggs128):
    return pl.pallas_call(
        _tc2_body,
        out_shape=jax.ShapeDtypeStruct((NH, 2 * H), jnp.float32),
    )(wa128, aggs128)


def _sc_dec_body(zq_hbm, za_hbm, qi_hbm, ai_hbm, pred_hbm,
                 qi_v, ai_v,
                 q0, q1, q2, q3, q4, a0, a1, a2, a3, a4,
                 tmp_v, out_v,
                 sq0, sq1, sq2, sq3, sq4, sa0, sa1, sa2, sa3, sa4):
    cid = lax.axis_index("c")
    sid = lax.axis_index("s")
    wid = sid * NC + cid
    base = wid * EPW
    qrs = (q0, q1, q2, q3, q4)
    ars = (a0, a1, a2, a3, a4)
    qsem = (sq0, sq1, sq2, sq3, sq4)
    asem = (sa0, sa1, sa2, sa3, sa4)

    pltpu.sync_copy(qi_hbm.at[wid], qi_v)
    pltpu.sync_copy(ai_hbm.at[wid], ai_v)

    def issue_gather(g, b):
        sl = pl.ds(g * BLK, BLK)
        pltpu.async_copy(zq_hbm.at[qi_v.at[sl]], qrs[b], qsem[b])
        pltpu.async_copy(za_hbm.at[ai_v.at[sl]], ars[b], asem[b])

    for b in range(3):
        issue_gather(b, b)

    @pl.loop(0, NBLK, step=NBUF)
    def _(blk0):
        for b in range(NBUF):
            g = blk0 + b
            sl = pl.ds(g * BLK, BLK)
            pltpu.make_async_copy(zq_hbm.at[qi_v.at[sl]],
                                  qrs[b], qsem[b]).wait()
            pltpu.make_async_copy(za_hbm.at[ai_v.at[sl]],
                                  ars[b], asem[b]).wait()

            @plsc.parallel_loop(0, BLK, unroll=4)
            def _(i):
                acc = qrs[b][i, pl.ds(0, L)] * ars[b][i, pl.ds(0, L)]
                for j in range(1, H // L):
                    slj = pl.ds(j * L, L)
                    acc = acc + qrs[b][i, slj] * ars[b][i, slj]
                tmp_v[pl.ds(i * L, L)] = acc

            iota = lax.iota(jnp.int32, L)
            for grp in range(BLK // L):
                res = jnp.zeros((L,), jnp.float32)
                for k in range(L):
                    idx = iota * L + (grp * L * L + k)
                    res = res + plsc.load_gather(tmp_v, [idx])
                res = 1.0 / (1.0 + jnp.exp(-res))
                out_v[pl.ds(grp * L, L)] = res

            pltpu.sync_copy(out_v, pred_hbm.at[pl.ds(base + g * BLK, BLK)])

            h = g + 3

            @pl.when(h < NBLK)
            def _():
                issue_gather(h, (b + 3) % NBUF)


def _sc_dec(zq, za, qi, ai):
    kern = functools.partial(
        pl.kernel,
        out_type=jax.ShapeDtypeStruct((N_EDGES,), jnp.float32),
        mesh=_vmesh,
        scratch_types=(
            [pltpu.VMEM((EPW,), jnp.int32),
             pltpu.VMEM((EPW,), jnp.int32)]
            + [pltpu.VMEM((BLK, H), jnp.float32) for _ in range(2 * NBUF)]
            + [pltpu.VMEM((BLK * L,), jnp.float32),
               pltpu.VMEM((BLK,), jnp.float32)]
            + [pltpu.SemaphoreType.DMA for _ in range(2 * NBUF)]
        ),
        compiler_params=_sc_params,
    )(_sc_dec_body)
    return kern(zq, za, qi, ai)


def _renumber(i):
    return jnp.where(i < NH, 2 * i, 2 * i - (N_NODES - 1))


def kernel(x_question, x_answer, edge_index, edge_label_index,
           Wq, bq, Wa, ba, W_lin, b_lin, edge_weight_logits):
    lg2d = edge_weight_logits.reshape(N_EDGES // D_IN, D_IN)

    t128, zq128, wa128, ew2d = _tc1(
        x_question, x_answer, Wq, bq.reshape(1, H), Wa, ba.reshape(1, H),
        W_lin, b_lin.reshape(1, H), lg2d)
    ew = ew2d.reshape(NW, EPW)

    src = _renumber(edge_index[0]).reshape(NW, EPW)
    dst = _renumber(edge_index[1]).reshape(NW, NBLK, BLK)
    zeros = jnp.zeros((N_PAD, H), jnp.float32)
    aggs = _sc_msg(t128.reshape(N_NODES, H), src, dst, ew, zeros)

    za128 = _tc2(wa128, aggs.reshape(NC, N_PAD // 2, 2 * H))

    return _sc_dec(zq128.reshape(N_NODES, H), za128.reshape(N_NODES, H),
                   _renumber(edge_label_index[0]).reshape(NW, EPW),
                   _renumber(edge_label_index[1]).reshape(NW, EPW))


# speedup vs baseline: 1.2161x; 1.1549x over previous
import dataclasses
import functools

import jax
import jax.numpy as jnp
from jax import lax
from jax.experimental import pallas as pl
from jax.experimental.pallas import tpu as pltpu
from jax.experimental.pallas import tpu_sc as plsc

N_NODES = 10000
N_EDGES = 320000
D_IN = 128
H = 64

NC = 2
NS = 16
L = 16
NW = NC * NS
EPW = N_EDGES // NW
BLK = 80
N_PAD = 10112
STRIPE = N_PAD // NS

_vmesh = plsc.VectorSubcoreMesh(
    core_axis_name="c", subcore_axis_name="s", num_cores=NC, num_subcores=NS)

_sc_params = pltpu.CompilerParams(
    needs_layout_passes=False, use_tc_tiling_on_sc=False)


NH = N_NODES // 2


def _tc1_body(xq_ref, xa_ref, wq_ref, bq_ref, wa_ref, ba_ref,
              wl_ref, bl_ref, lg_ref,
              t_ref, zq_ref, wa_out_ref, ew_ref):
    wq = wq_ref[...]
    wa = wa_ref[...]
    wl = wl_ref[...]
    z64 = jnp.zeros((H, H), jnp.float32)
    wl2 = jnp.concatenate(
        [jnp.concatenate([wl, z64], axis=1),
         jnp.concatenate([z64, wl], axis=1)], axis=0)
    bb = jnp.concatenate([bl_ref[...], bl_ref[...]], axis=1)
    h_qt = jnp.maximum(jnp.dot(xq_ref[0:NH, :], wq,
                               preferred_element_type=jnp.float32)
                       + bq_ref[...], 0.0)
    h_qb = jnp.maximum(jnp.dot(xq_ref[NH:, :], wq,
                               preferred_element_type=jnp.float32)
                       + bq_ref[...], 0.0)
    h128 = jnp.concatenate([h_qt, h_qb], axis=1)
    t128 = jnp.dot(h128, wl2, preferred_element_type=jnp.float32)
    t_ref[...] = t128
    zq_ref[...] = t128 + bb

    h_at = jnp.maximum(jnp.dot(xa_ref[0:NH, :], wa,
                               preferred_element_type=jnp.float32)
                       + ba_ref[...], 0.0)
    h_ab = jnp.maximum(jnp.dot(xa_ref[NH:, :], wa,
                               preferred_element_type=jnp.float32)
                       + ba_ref[...], 0.0)
    a128 = jnp.concatenate([h_at, h_ab], axis=1)
    wa_out_ref[...] = (jnp.dot(a128, wl2,
                               preferred_element_type=jnp.float32)
                       + bb)

    ew_ref[...] = jax.nn.sigmoid(lg_ref[...])


def _tc1(xq, xa, wq, bq, wa, ba, wl, bl, lg2d):
    return pl.pallas_call(
        _tc1_body,
        out_shape=(
            jax.ShapeDtypeStruct((NH, 2 * H), jnp.float32),
            jax.ShapeDtypeStruct((NH, 2 * H), jnp.float32),
            jax.ShapeDtypeStruct((NH, 2 * H), jnp.float32),
            jax.ShapeDtypeStruct(lg2d.shape, jnp.float32),
        ),
    )(xq, xa, wq, bq, wa, ba, wl, bl, lg2d)


NBUF = 5
NBLK = EPW // BLK


def _sc_msg_body(hq_hbm, src_hbm, dst_hbm, ew_hbm, zeros_hbm, agg_hbm,
                 src_v, dst_v, ew_v,
                 r0, r1, r2, r3, r4,
                 agg_sh,
                 g0s, g1s, g2s, g3s, g4s,
                 s0s, s1s, s2s, s3s, s4s):
    cid = lax.axis_index("c")
    sid = lax.axis_index("s")
    wid = sid * NC + cid
    rows = (r0, r1, r2, r3, r4)
    gsem = (g0s, g1s, g2s, g3s, g4s)
    ssem = (s0s, s1s, s2s, s3s, s4s)

    pltpu.sync_copy(zeros_hbm.at[pl.ds(sid * STRIPE, STRIPE)],
                    agg_sh.at[pl.ds(sid * STRIPE, STRIPE)])
    pltpu.sync_copy(src_hbm.at[wid], src_v)
    pltpu.sync_copy(dst_hbm.at[wid], dst_v)
    pltpu.sync_copy(ew_hbm.at[wid], ew_v)
    plsc.subcore_barrier()

    def issue_gather(g, b):
        pltpu.async_copy(hq_hbm.at[src_v.at[pl.ds(g * BLK, BLK)]],
                         rows[b], gsem[b])

    for b in range(3):
        issue_gather(b, b)

    @pl.loop(0, NBLK, step=NBUF)
    def _(blk0):
        for b in range(NBUF):
            g = blk0 + b
            pltpu.make_async_copy(hq_hbm.at[src_v.at[pl.ds(g * BLK, BLK)]],
                                  rows[b], gsem[b]).wait()

            @plsc.parallel_loop(0, BLK, unroll=4)
            def _(i):
                wv = plsc.load_gather(
                    ew_v, [jnp.full((L,), g * BLK + i, jnp.int32)])
                for j in range(H // L):
                    sl = pl.ds(j * L, L)
                    rows[b][i, sl] = rows[b][i, sl] * wv

            pltpu.async_copy(rows[b], agg_sh.at[dst_v.at[g]],
                             ssem[b], add=True)

            h = g + 3
            bh = (b + 3) % NBUF

            @pl.when(jnp.logical_and(h >= NBUF, h < NBLK))
            def _():
                pltpu.make_async_copy(rows[bh], agg_sh.at[dst_v.at[h]],
                                      ssem[bh]).wait()

            @pl.when(h < NBLK)
            def _():
                issue_gather(h, bh)

    for b in range(NBUF):
        pltpu.make_async_copy(rows[b], agg_sh.at[dst_v.at[0]], ssem[b]).wait()
    plsc.subcore_barrier()
    pltpu.sync_copy(agg_sh.at[pl.ds(sid * STRIPE, STRIPE)],
                    agg_hbm.at[cid].at[pl.ds(sid * STRIPE, STRIPE)])


def _sc_msg(hq, src, dst, ew, zeros):
    kern = functools.partial(
        pl.kernel,
        out_type=jax.ShapeDtypeStruct((NC, N_PAD, H), jnp.float32),
        mesh=_vmesh,
        scratch_types=(
            [pltpu.VMEM((EPW,), jnp.int32),
             pltpu.VMEM((NBLK, BLK), jnp.int32),
             pltpu.VMEM((EPW,), jnp.float32)]
            + [pltpu.VMEM((BLK, H), jnp.float32) for _ in range(NBUF)]
            + [pltpu.VMEM_SHARED((N_PAD, H), jnp.float32)]
            + [pltpu.SemaphoreType.DMA for _ in range(2 * NBUF)]
        ),
        compiler_params=_sc_params,
    )(_sc_msg_body)
    return kern(hq, src, dst, ew, zeros)


def _tc2_body(wa_ref, aggs_ref, za_ref):
    za_ref[...] = (wa_ref[...] + aggs_ref[0, 0:NH, :]
                   + aggs_ref[1, 0:NH, :])


def _tc2(wa128, aggs128):
    return pl.pallas_call(
        _tc2_body,
        out_shape=jax.ShapeDtypeStruct((NH, 2 * H), jnp.float32),
    )(wa128, aggs128)


def _sc_dec_body(zq_hbm, za_hbm, qi_hbm, ai_hbm, pred_hbm,
                 qi_v, ai_v,
                 q0, q1, q2, q3, q4, a0, a1, a2, a3, a4,
                 tmp_v, out_v,
                 sq0, sq1, sq2, sq3, sq4, sa0, sa1, sa2, sa3, sa4):
    cid = lax.axis_index("c")
    sid = lax.axis_index("s")
    wid = sid * NC + cid
    base = wid * EPW
    qrs = (q0, q1, q2, q3, q4)
    ars = (a0, a1, a2, a3, a4)
    qsem = (sq0, sq1, sq2, sq3, sq4)
    asem = (sa0, sa1, sa2, sa3, sa4)

    pltpu.sync_copy(qi_hbm.at[wid], qi_v)
    pltpu.sync_copy(ai_hbm.at[wid], ai_v)

    def issue_gather(g, b):
        sl = pl.ds(g * BLK, BLK)
        pltpu.async_copy(zq_hbm.at[qi_v.at[sl]], qrs[b], qsem[b])
        pltpu.async_copy(za_hbm.at[ai_v.at[sl]], ars[b], asem[b])

    for b in range(3):
        issue_gather(b, b)

    @pl.loop(0, NBLK, step=NBUF)
    def _(blk0):
        for b in range(NBUF):
            g = blk0 + b
            sl = pl.ds(g * BLK, BLK)
            pltpu.make_async_copy(zq_hbm.at[qi_v.at[sl]],
                                  qrs[b], qsem[b]).wait()
            pltpu.make_async_copy(za_hbm.at[ai_v.at[sl]],
                                  ars[b], asem[b]).wait()

            last_lane = lax.iota(jnp.int32, L) == (L - 1)

            @plsc.parallel_loop(0, BLK, unroll=4)
            def _(i):
                acc = qrs[b][i, pl.ds(0, L)] * ars[b][i, pl.ds(0, L)]
                for j in range(1, H // L):
                    slj = pl.ds(j * L, L)
                    acc = acc + qrs[b][i, slj] * ars[b][i, slj]
                tot = plsc.cumsum(acc)
                plsc.store_scatter(out_v, [jnp.full((L,), i, jnp.int32)],
                                   tot, mask=last_lane)

            for grp in range(BLK // L):
                sl = pl.ds(grp * L, L)
                out_v[sl] = 1.0 / (1.0 + jnp.exp(-out_v[sl]))

            pltpu.sync_copy(out_v, pred_hbm.at[pl.ds(base + g * BLK, BLK)])

            h = g + 3

            @pl.when(h < NBLK)
            def _():
                issue_gather(h, (b + 3) % NBUF)


def _sc_dec(zq, za, qi, ai):
    kern = functools.partial(
        pl.kernel,
        out_type=jax.ShapeDtypeStruct((N_EDGES,), jnp.float32),
        mesh=_vmesh,
        scratch_types=(
            [pltpu.VMEM((EPW,), jnp.int32),
             pltpu.VMEM((EPW,), jnp.int32)]
            + [pltpu.VMEM((BLK, H), jnp.float32) for _ in range(2 * NBUF)]
            + [pltpu.VMEM((BLK * L,), jnp.float32),
               pltpu.VMEM((BLK,), jnp.float32)]
            + [pltpu.SemaphoreType.DMA for _ in range(2 * NBUF)]
        ),
        compiler_params=_sc_params,
    )(_sc_dec_body)
    return kern(zq, za, qi, ai)


def _renumber(i):
    return jnp.where(i < NH, 2 * i, 2 * i - (N_NODES - 1))


def kernel(x_question, x_answer, edge_index, edge_label_index,
           Wq, bq, Wa, ba, W_lin, b_lin, edge_weight_logits):
    lg2d = edge_weight_logits.reshape(N_EDGES // D_IN, D_IN)

    t128, zq128, wa128, ew2d = _tc1(
        x_question, x_answer, Wq, bq.reshape(1, H), Wa, ba.reshape(1, H),
        W_lin, b_lin.reshape(1, H), lg2d)
    ew = ew2d.reshape(NW, EPW)

    src = _renumber(edge_index[0]).reshape(NW, EPW)
    dst = _renumber(edge_index[1]).reshape(NW, NBLK, BLK)
    zeros = jnp.zeros((N_PAD, H), jnp.float32)
    aggs = _sc_msg(t128.reshape(N_NODES, H), src, dst, ew, zeros)

    za128 = _tc2(wa128, aggs.reshape(NC, N_PAD // 2, 2 * H))

    return _sc_dec(zq128.reshape(N_NODES, H), za128.reshape(N_NODES, H),
                   _renumber(edge_label_index[0]).reshape(NW, EPW),
                   _renumber(edge_label_index[1]).reshape(NW, EPW))
